# baseline (speedup 1.0000x reference)
import jax
import jax.numpy as jnp
from jax import lax
from jax.experimental import pallas as pl
from jax.experimental.pallas import tpu as pltpu
from jax.experimental.pallas import tpu_sc as plsc

_NUM_FIELDS = 26
_VOCAB_PER_FIELD = 100000
_EMBED_DIM = 128
_BATCH = 4096
_NC = 2
_NS = 16
_NW = _NC * _NS
_BPW = _BATCH // _NW
_NBUF = 14


def _fuse_row(idx_v, f):
    off = f * _VOCAB_PER_FIELD
    for k in range(_BPW // 16):
        idx_v[f, pl.ds(k * 16, 16)] = idx_v[f, pl.ds(k * 16, 16)] + off


def _sc_body(inp_hbm, table_hbm, out_hbm, idx_v, rows_v, *sems):
    wid = lax.axis_index("s") * _NC + lax.axis_index("c")
    b0 = wid * _BPW
    gsems = sems[:_NBUF]
    osems = sems[_NBUF:]

    pltpu.sync_copy(inp_hbm.at[:, pl.ds(b0, _BPW)], idx_v)

    def gather_copy(j, b):
        f, h = j // 2, (j % 2) * 64
        return pltpu.make_async_copy(
            table_hbm.at[idx_v.at[f, pl.ds(h, 64)]], rows_v.at[b], gsems[b])

    def out_copy(j, b):
        f, h = j // 2, (j % 2) * 64
        return pltpu.make_async_copy(
            rows_v.at[b], out_hbm.at[f, pl.ds(b0 + h, 64)], osems[b])

    for b in range(_NBUF):
        if b % 2 == 0:
            _fuse_row(idx_v, b // 2)
        gather_copy(b, b).start()
    for f in range(_NBUF // 2, _NUM_FIELDS):
        _fuse_row(idx_v, f)

    nch = _NUM_FIELDS * 2
    for j in range(nch):
        b = j % _NBUF
        gather_copy(j, b).wait()
        out_copy(j, b).start()
        jn = j + _NBUF
        if jn < nch:
            out_copy(j, b).wait()
            gather_copy(jn, b).start()
    for j in range(nch - _NBUF, nch):
        out_copy(j, j % _NBUF).wait()


def kernel(inputs, table):
    inp_t = inputs.T
    mesh = plsc.VectorSubcoreMesh(core_axis_name="c", subcore_axis_name="s")
    run = pl.kernel(
        _sc_body,
        out_type=jax.ShapeDtypeStruct((_NUM_FIELDS, _BATCH, _EMBED_DIM),
                                      jnp.float32),
        mesh=mesh,
        scratch_types=[
            pltpu.VMEM((_NUM_FIELDS, _BPW), jnp.int32),
            pltpu.VMEM((_NBUF, 64, _EMBED_DIM), jnp.float32),
        ] + [pltpu.SemaphoreType.DMA] * (2 * _NBUF),
    )
    out = run(inp_t, table)
    return out.transpose(1, 0, 2)


# speedup vs baseline: 1.0200x; 1.0200x over previous
import jax
import jax.numpy as jnp
from jax import lax
from jax.experimental import pallas as pl
from jax.experimental.pallas import tpu as pltpu
from jax.experimental.pallas import tpu_sc as plsc

_NUM_FIELDS = 26
_VOCAB_PER_FIELD = 100000
_EMBED_DIM = 128
_BATCH = 4096
_NC = 2
_NS = 16
_NW = _NC * _NS
_BPW = _BATCH // _NW
_NBUF = 7


def _fuse_row(idx_v, f):
    off = f * _VOCAB_PER_FIELD
    for k in range(_BPW // 16):
        idx_v[f, pl.ds(k * 16, 16)] = idx_v[f, pl.ds(k * 16, 16)] + off


def _sc_body(inp_hbm, table_hbm, out_hbm, idx_v, rows_v, *sems):
    wid = lax.axis_index("s") * _NC + lax.axis_index("c")
    b0 = wid * _BPW
    gsems = sems[:_NBUF]
    osems = sems[_NBUF:]

    pltpu.sync_copy(inp_hbm.at[:, pl.ds(b0, _BPW)], idx_v)

    def gather_copy(f, b):
        return pltpu.make_async_copy(
            table_hbm.at[idx_v.at[f]], rows_v.at[b], gsems[b])

    def out_copy(f, b):
        return pltpu.make_async_copy(
            rows_v.at[b], out_hbm.at[f, pl.ds(b0, _BPW)], osems[b])

    for b in range(_NBUF):
        _fuse_row(idx_v, b)
        gather_copy(b, b).start()
    for f in range(_NBUF, _NUM_FIELDS):
        _fuse_row(idx_v, f)

    for f in range(_NUM_FIELDS):
        b = f % _NBUF
        gather_copy(f, b).wait()
        out_copy(f, b).start()
        fn = f + _NBUF
        if fn < _NUM_FIELDS:
            out_copy(f, b).wait()
            gather_copy(fn, b).start()
    for f in range(_NUM_FIELDS - _NBUF, _NUM_FIELDS):
        out_copy(f, f % _NBUF).wait()


def kernel(inputs, table):
    inp_t = inputs.T
    mesh = plsc.VectorSubcoreMesh(core_axis_name="c", subcore_axis_name="s")
    run = pl.kernel(
        _sc_body,
        out_type=jax.ShapeDtypeStruct((_NUM_FIELDS, _BATCH, _EMBED_DIM),
                                      jnp.float32),
        mesh=mesh,
        scratch_types=[
            pltpu.VMEM((_NUM_FIELDS, _BPW), jnp.int32),
            pltpu.VMEM((_NBUF, _BPW, _EMBED_DIM), jnp.float32),
        ] + [pltpu.SemaphoreType.DMA] * (2 * _NBUF),
    )
    out = run(inp_t, table)
    return out.transpose(1, 0, 2)


# speedup vs baseline: 1.0201x; 1.0001x over previous
import jax
import jax.numpy as jnp
from jax import lax
from jax.experimental import pallas as pl
from jax.experimental.pallas import tpu as pltpu
from jax.experimental.pallas import tpu_sc as plsc

_NUM_FIELDS = 26
_VOCAB_PER_FIELD = 100000
_EMBED_DIM = 128
_BATCH = 4096
_NC = 2
_NS = 16
_NW = _NC * _NS
_BPW = _BATCH // _NW
_NBUF = 7


def _fuse_row(idx_v, f):
    off = f * _VOCAB_PER_FIELD
    for k in range(_BPW // 16):
        idx_v[f, pl.ds(k * 16, 16)] = idx_v[f, pl.ds(k * 16, 16)] + off


def _sc_body(inp_hbm, table_hbm, out_hbm, idx_v, rows_v, *sems):
    wid = lax.axis_index("s") * _NC + lax.axis_index("c")
    b0 = wid * _BPW
    gsems = sems[:_NBUF]
    osems = sems[_NBUF:]

    _SPLIT = 8
    s1 = pltpu.make_async_copy(inp_hbm.at[pl.ds(0, _SPLIT), pl.ds(b0, _BPW)],
                               idx_v.at[pl.ds(0, _SPLIT)], osems[0])
    s2 = pltpu.make_async_copy(
        inp_hbm.at[pl.ds(_SPLIT, _NUM_FIELDS - _SPLIT), pl.ds(b0, _BPW)],
        idx_v.at[pl.ds(_SPLIT, _NUM_FIELDS - _SPLIT)], osems[1])
    s1.start()
    s2.start()
    s1.wait()

    def gather_copy(f, b):
        return pltpu.make_async_copy(
            table_hbm.at[idx_v.at[f]], rows_v.at[b], gsems[b])

    def out_copy(f, b):
        return pltpu.make_async_copy(
            rows_v.at[b], out_hbm.at[f, pl.ds(b0, _BPW)], osems[b])

    for b in range(_NBUF):
        _fuse_row(idx_v, b)
        gather_copy(b, b).start()
    s2.wait()
    for f in range(_NBUF, _NUM_FIELDS):
        _fuse_row(idx_v, f)

    for f in range(_NUM_FIELDS):
        b = f % _NBUF
        gather_copy(f, b).wait()
        out_copy(f, b).start()
        fn = f + _NBUF
        if fn < _NUM_FIELDS:
            out_copy(f, b).wait()
            gather_copy(fn, b).start()
    for f in range(_NUM_FIELDS - _NBUF, _NUM_FIELDS):
        out_copy(f, f % _NBUF).wait()


def kernel(inputs, table):
    inp_t = inputs.T
    mesh = plsc.VectorSubcoreMesh(core_axis_name="c", subcore_axis_name="s")
    run = pl.kernel(
        _sc_body,
        out_type=jax.ShapeDtypeStruct((_NUM_FIELDS, _BATCH, _EMBED_DIM),
                                      jnp.float32),
        mesh=mesh,
        scratch_types=[
            pltpu.VMEM((_NUM_FIELDS, _BPW), jnp.int32),
            pltpu.VMEM((_NBUF, _BPW, _EMBED_DIM), jnp.float32),
        ] + [pltpu.SemaphoreType.DMA] * (2 * _NBUF),
    )
    out = run(inp_t, table)
    return out.transpose(1, 0, 2)
